# baseline (speedup 1.0000x reference)
import functools

import jax
import jax.numpy as jnp
from jax.experimental import pallas as pl
from jax.experimental.pallas import tpu as pltpu

_K = 30
_CPB = 6
_NEG = float("-inf")
_BIG = 2**30


def _fused_kernel(x_ref, w_ref, b_ref, o_ref, scores_ref):
    H, W = x_ref.shape[2], x_ref.shape[3]
    phase = pl.program_id(2)

    @pl.when(phase == 0)
    def _conv():
        zcol = jnp.zeros((H, 1), jnp.float32)
        zrow = jnp.zeros((1, W), jnp.float32)
        for i in range(_CPB):
            xm = x_ref[0, i].astype(jnp.bfloat16).astype(jnp.float32)
            xl = jnp.concatenate([zcol, xm[:, : W - 1]], axis=1)
            xr = jnp.concatenate([xm[:, 1:], zcol], axis=1)

            def sd(v):
                return jnp.concatenate([zrow, v[: H - 1, :]], axis=0)

            def su(v):
                return jnp.concatenate([v[1:, :], zrow], axis=0)

            w = [w_ref[i, 0, j] for j in range(9)]
            scores_ref[i] = (
                w[0] * sd(xl) + w[1] * sd(xm) + w[2] * sd(xr)
                + w[3] * xl + w[4] * xm + w[5] * xr
                + w[6] * su(xl) + w[7] * su(xm) + w[8] * su(xr)
                + b_ref[i, 0, 0]
            )

    @pl.when(phase == 1)
    def _select():
        G = H // 8
        piota = (jax.lax.broadcasted_iota(jnp.int32, (G, 8), 0) * 8
                 + jax.lax.broadcasted_iota(jnp.int32, (G, 8), 1))
        citota = jax.lax.broadcasted_iota(jnp.int32, (1, W), 1)

        rms = [jnp.max(scores_ref[i].reshape(G, 8, W), axis=2)
               for i in range(_CPB)]
        rows = [[] for _ in range(_CPB)]
        cand_rows = [[] for _ in range(_CPB)]
        gidx_rows = [[] for _ in range(_CPB)]
        for k in range(_K):
            for i in range(_CPB):
                m = jnp.max(rms[i])
                r = jnp.min(jnp.where(rms[i] == m, piota, _BIG))
                rows[i].append(r)
                cand_rows[i].append(scores_ref[i, pl.ds(r, 1), :])
                gidx_rows[i].append(r * W + citota)
                rms[i] = jnp.where(piota == r, _NEG, rms[i])

        cands = [jnp.concatenate(cand_rows[i], axis=0) for i in range(_CPB)]
        gidxs = [jnp.concatenate(gidx_rows[i], axis=0) for i in range(_CPB)]
        msks = [jnp.zeros((_K, W), jnp.bool_) for _ in range(_CPB)]
        for k in range(_K):
            for i in range(_CPB):
                m = jnp.max(cands[i])
                sel = jnp.min(jnp.where(cands[i] == m, gidxs[i], _BIG))
                msks[i] = msks[i] | (gidxs[i] == sel)
                cands[i] = jnp.where(gidxs[i] == sel, _NEG, cands[i])

        for i in range(_CPB):
            o_ref[0, i] = jnp.zeros((H, W), jnp.float32)
        for k in range(_K):
            for i in range(_CPB):
                r = rows[i][k]
                xrow = x_ref[0, i, pl.ds(r, 1), :]
                o_ref[0, i, pl.ds(r, 1), :] = jnp.where(
                    msks[i][k : k + 1, :], xrow, 0.0)


@jax.jit
def kernel(x, conv_w, conv_b):
    B, C, H, W = x.shape
    wmat = conv_w.reshape(C, 1, 9)
    bias = conv_b.reshape(C, 1, 1)
    return pl.pallas_call(
        _fused_kernel,
        grid=(B, C // _CPB, 2),
        in_specs=[
            pl.BlockSpec((1, _CPB, H, W), lambda b, c, p: (b, c, 0, 0)),
            pl.BlockSpec((_CPB, 1, 9), lambda b, c, p: (c, 0, 0)),
            pl.BlockSpec((_CPB, 1, 1), lambda b, c, p: (c, 0, 0)),
        ],
        out_specs=pl.BlockSpec((1, _CPB, H, W), lambda b, c, p: (b, c, 0, 0)),
        out_shape=jax.ShapeDtypeStruct((B, C, H, W), jnp.float32),
        scratch_shapes=[
            pltpu.VMEM((_CPB, H, W), jnp.float32),
        ],
        compiler_params=pltpu.CompilerParams(
            dimension_semantics=("arbitrary", "arbitrary", "arbitrary"),
        ),
    )(x, wmat, bias)


# speedup vs baseline: 1.3430x; 1.3430x over previous
import functools

import jax
import jax.numpy as jnp
from jax.experimental import pallas as pl
from jax.experimental.pallas import tpu as pltpu

_K = 30
_CPB = 4
_NEG = float("-inf")
_BIG = 2**30


def _fused_kernel(x_ref, xp_ref, w_ref, b_ref, o_ref, scores_ref):
    H, W = x_ref.shape[2], x_ref.shape[3]
    s = pl.program_id(0)
    nchunks = pl.num_programs(0) - 1

    @pl.when(s < nchunks)
    def _conv():
        buf = jax.lax.rem(s, 2)
        zcol = jnp.zeros((H, 1), jnp.float32)
        zrow = jnp.zeros((1, W), jnp.float32)
        for i in range(_CPB):
            xm = x_ref[0, i].astype(jnp.bfloat16).astype(jnp.float32)
            xl = jnp.concatenate([zcol, xm[:, : W - 1]], axis=1)
            xr = jnp.concatenate([xm[:, 1:], zcol], axis=1)

            def sd(v):
                return jnp.concatenate([zrow, v[: H - 1, :]], axis=0)

            def su(v):
                return jnp.concatenate([v[1:, :], zrow], axis=0)

            w = [w_ref[i, 0, j] for j in range(9)]
            scores_ref[buf, i] = (
                w[0] * sd(xl) + w[1] * sd(xm) + w[2] * sd(xr)
                + w[3] * xl + w[4] * xm + w[5] * xr
                + w[6] * su(xl) + w[7] * su(xm) + w[8] * su(xr)
                + b_ref[i, 0, 0]
            )

    @pl.when(s > 0)
    def _select():
        buf = jax.lax.rem(s + 1, 2)
        riota = jax.lax.broadcasted_iota(jnp.int32, (H, 1), 0)
        citota = jax.lax.broadcasted_iota(jnp.int32, (1, W), 1)

        rms = [jnp.max(scores_ref[buf, i], axis=1, keepdims=True)
               for i in range(_CPB)]
        rows = [[] for _ in range(_CPB)]
        cand_rows = [[] for _ in range(_CPB)]
        gidx_rows = [[] for _ in range(_CPB)]
        for k in range(_K):
            for i in range(_CPB):
                m = jnp.max(rms[i])
                r = jnp.min(jnp.where(rms[i] == m, riota, _BIG))
                rows[i].append(r)
                cand_rows[i].append(scores_ref[buf, i, pl.ds(r, 1), :])
                gidx_rows[i].append(r * W + citota)
                rms[i] = jnp.where(riota == r, _NEG, rms[i])

        cands = [jnp.concatenate(cand_rows[i], axis=0) for i in range(_CPB)]
        gidxs = [jnp.concatenate(gidx_rows[i], axis=0) for i in range(_CPB)]
        msks = [jnp.zeros((_K, W), jnp.bool_) for _ in range(_CPB)]
        for k in range(_K):
            for i in range(_CPB):
                m = jnp.max(cands[i])
                sel = jnp.min(jnp.where(cands[i] == m, gidxs[i], _BIG))
                msks[i] = msks[i] | (gidxs[i] == sel)
                cands[i] = jnp.where(gidxs[i] == sel, _NEG, cands[i])

        for i in range(_CPB):
            o_ref[0, i] = jnp.zeros((H, W), jnp.float32)
        for k in range(_K):
            for i in range(_CPB):
                r = rows[i][k]
                xrow = xp_ref[0, i, pl.ds(r, 1), :]
                o_ref[0, i, pl.ds(r, 1), :] = jnp.where(
                    msks[i][k : k + 1, :], xrow, 0.0)


@jax.jit
def kernel(x, conv_w, conv_b):
    B, C, H, W = x.shape
    CC = C // _CPB
    NC = B * CC
    wmat = conv_w.reshape(C, 1, 9)
    bias = conv_b.reshape(C, 1, 1)

    def cur(s):
        t = jnp.minimum(s, NC - 1)
        return t // CC, t % CC

    def prev(s):
        t = jnp.maximum(s - 1, 0)
        return t // CC, t % CC

    return pl.pallas_call(
        _fused_kernel,
        grid=(NC + 1,),
        in_specs=[
            pl.BlockSpec((1, _CPB, H, W), lambda s: (*cur(s), 0, 0)),
            pl.BlockSpec((1, _CPB, H, W), lambda s: (*prev(s), 0, 0)),
            pl.BlockSpec((_CPB, 1, 9), lambda s: (cur(s)[1], 0, 0)),
            pl.BlockSpec((_CPB, 1, 1), lambda s: (cur(s)[1], 0, 0)),
        ],
        out_specs=pl.BlockSpec((1, _CPB, H, W), lambda s: (*prev(s), 0, 0)),
        out_shape=jax.ShapeDtypeStruct((B, C, H, W), jnp.float32),
        scratch_shapes=[
            pltpu.VMEM((2, _CPB, H, W), jnp.float32),
        ],
        compiler_params=pltpu.CompilerParams(
            dimension_semantics=("arbitrary",),
        ),
    )(x, x, wmat, bias)


# speedup vs baseline: 1.3453x; 1.0017x over previous
import functools

import jax
import jax.numpy as jnp
from jax.experimental import pallas as pl
from jax.experimental.pallas import tpu as pltpu

_K = 30
_CPB = 6
_NEG = float("-inf")
_BIG = 2**30


def _fused_kernel(x_ref, w_ref, b_ref, o_ref, scores_ref):
    H, W = x_ref.shape[2], x_ref.shape[3]
    phase = pl.program_id(2)

    @pl.when(phase == 0)
    def _conv():
        zcol = jnp.zeros((H, 1), jnp.float32)
        zrow = jnp.zeros((1, W), jnp.float32)
        for i in range(_CPB):
            xm = x_ref[0, i].astype(jnp.bfloat16).astype(jnp.float32)
            xl = jnp.concatenate([zcol, xm[:, : W - 1]], axis=1)
            xr = jnp.concatenate([xm[:, 1:], zcol], axis=1)

            def sd(v):
                return jnp.concatenate([zrow, v[: H - 1, :]], axis=0)

            def su(v):
                return jnp.concatenate([v[1:, :], zrow], axis=0)

            w = [w_ref[i, 0, j] for j in range(9)]
            scores_ref[i] = (
                w[0] * sd(xl) + w[1] * sd(xm) + w[2] * sd(xr)
                + w[3] * xl + w[4] * xm + w[5] * xr
                + w[6] * su(xl) + w[7] * su(xm) + w[8] * su(xr)
                + b_ref[i, 0, 0]
            )

    @pl.when(phase == 1)
    def _select():
        riota = jax.lax.broadcasted_iota(jnp.int32, (H, 1), 0)
        citota = jax.lax.broadcasted_iota(jnp.int32, (1, W), 1)

        rms = [jnp.max(scores_ref[i], axis=1, keepdims=True) for i in range(_CPB)]
        rows = [[] for _ in range(_CPB)]
        cand_rows = [[] for _ in range(_CPB)]
        gidx_rows = [[] for _ in range(_CPB)]
        for k in range(_K):
            for i in range(_CPB):
                m = jnp.max(rms[i])
                r = jnp.min(jnp.where(rms[i] == m, riota, _BIG))
                rows[i].append(r)
                cand_rows[i].append(scores_ref[i, pl.ds(r, 1), :])
                gidx_rows[i].append(r * W + citota)
                rms[i] = jnp.where(riota == r, _NEG, rms[i])

        cands = [jnp.concatenate(cand_rows[i], axis=0) for i in range(_CPB)]
        gidxs = [jnp.concatenate(gidx_rows[i], axis=0) for i in range(_CPB)]
        msks = [jnp.zeros((_K, W), jnp.bool_) for _ in range(_CPB)]
        for k in range(_K):
            for i in range(_CPB):
                m = jnp.max(cands[i])
                sel = jnp.min(jnp.where(cands[i] == m, gidxs[i], _BIG))
                msks[i] = msks[i] | (gidxs[i] == sel)
                cands[i] = jnp.where(gidxs[i] == sel, _NEG, cands[i])

        for i in range(_CPB):
            o_ref[0, i] = jnp.zeros((H, W), jnp.float32)
        for k in range(_K):
            for i in range(_CPB):
                r = rows[i][k]
                xrow = x_ref[0, i, pl.ds(r, 1), :]
                o_ref[0, i, pl.ds(r, 1), :] = jnp.where(
                    msks[i][k : k + 1, :], xrow, 0.0)


@jax.jit
def kernel(x, conv_w, conv_b):
    B, C, H, W = x.shape
    wmat = conv_w.reshape(C, 1, 9)
    bias = conv_b.reshape(C, 1, 1)
    return pl.pallas_call(
        _fused_kernel,
        grid=(B, C // _CPB, 2),
        in_specs=[
            pl.BlockSpec((1, _CPB, H, W), lambda b, c, p: (b, c, 0, 0)),
            pl.BlockSpec((_CPB, 1, 9), lambda b, c, p: (c, 0, 0)),
            pl.BlockSpec((_CPB, 1, 1), lambda b, c, p: (c, 0, 0)),
        ],
        out_specs=pl.BlockSpec((1, _CPB, H, W), lambda b, c, p: (b, c, 0, 0)),
        out_shape=jax.ShapeDtypeStruct((B, C, H, W), jnp.float32),
        scratch_shapes=[
            pltpu.VMEM((_CPB, H, W), jnp.float32),
        ],
        compiler_params=pltpu.CompilerParams(
            dimension_semantics=("arbitrary", "arbitrary", "arbitrary"),
        ),
    )(x, wmat, bias)
